# baseline (speedup 1.0000x reference)
import jax
import jax.numpy as jnp
from jax.experimental import pallas as pl
from jax.experimental.pallas import tpu as pltpu

CODEBOOK = 8192
DIM = 32
TILE = 256
CHUNK = 4096
NCHUNK = CODEBOOK // CHUNK


def _vq_tile_kernel(x_ref, x2_ref, ehi_ref, elo_ref, e2_ref, emb_ref,
                    quant_ref, idx_ref, loss_ref):
    xt = x_ref[...]
    x_bf = xt.astype(jnp.bfloat16)
    x2 = x2_ref[...]

    v = jnp.full((TILE, 1), jnp.inf, jnp.float32)
    ix = jnp.zeros((TILE, 1), jnp.int32)
    for c in range(NCHUNK):
        s = slice(c * CHUNK, (c + 1) * CHUNK)
        mm = (jax.lax.dot_general(
                  x_bf, ehi_ref[:, s], (((1,), (0,)), ((), ())),
                  preferred_element_type=jnp.float32)
              + jax.lax.dot_general(
                  x_bf, elo_ref[:, s], (((1,), (0,)), ((), ())),
                  preferred_element_type=jnp.float32))
        d2 = (x2 + e2_ref[:, s]) - 2.0 * mm
        dist = jnp.sqrt(jnp.maximum(d2, 0.0))
        cmin = jnp.min(dist, axis=1, keepdims=True)
        lane = jax.lax.broadcasted_iota(jnp.int32, dist.shape, 1) + c * CHUNK
        cidx = jnp.min(jnp.where(dist == cmin, lane, CODEBOOK),
                       axis=1, keepdims=True)
        take = cmin < v
        v = jnp.where(take, cmin, v)
        ix = jnp.where(take, cidx, ix)
        v = v.astype(jnp.bfloat16).astype(jnp.float32)

    idx_ref[...] = ix
    lane_full = jax.lax.broadcasted_iota(jnp.int32, (TILE, CODEBOOK), 1)
    onehot = (lane_full == ix).astype(jnp.float32)
    quant = jax.lax.dot_general(
        onehot, emb_ref[...], (((1,), (0,)), ((), ())),
        preferred_element_type=jnp.float32)
    quant_ref[...] = quant
    diff = quant - xt
    part = jnp.sum(diff * diff)

    @pl.when(pl.program_id(0) == 0)
    def _():
        loss_ref[0, 0] = 0.0

    loss_ref[0, 0] += part


@jax.jit
def kernel(x, emb):
    B, C, H, W = x.shape
    n = B * H * W
    xf = jnp.transpose(x, (0, 2, 3, 1)).reshape(-1, C)
    x2 = jnp.sum(xf ** 2, axis=1, keepdims=True)
    e2 = jnp.sum(emb ** 2, axis=1)[None, :]
    ehi = emb.astype(jnp.bfloat16)
    elo = (emb - ehi.astype(jnp.float32)).astype(jnp.bfloat16)
    ehi_t = ehi.T
    elo_t = elo.T

    grid = n // TILE
    quant, idx, loss = pl.pallas_call(
        _vq_tile_kernel,
        grid=(grid,),
        in_specs=[
            pl.BlockSpec((TILE, C), lambda i: (i, 0)),
            pl.BlockSpec((TILE, 1), lambda i: (i, 0)),
            pl.BlockSpec((C, CODEBOOK), lambda i: (0, 0)),
            pl.BlockSpec((C, CODEBOOK), lambda i: (0, 0)),
            pl.BlockSpec((1, CODEBOOK), lambda i: (0, 0)),
            pl.BlockSpec((CODEBOOK, C), lambda i: (0, 0)),
        ],
        out_specs=[
            pl.BlockSpec((TILE, C), lambda i: (i, 0)),
            pl.BlockSpec((TILE, 1), lambda i: (i, 0)),
            pl.BlockSpec((1, 1), lambda i: (0, 0),
                         memory_space=pltpu.SMEM),
        ],
        out_shape=[
            jax.ShapeDtypeStruct((n, C), jnp.float32),
            jax.ShapeDtypeStruct((n, 1), jnp.int32),
            jax.ShapeDtypeStruct((1, 1), jnp.float32),
        ],
    )(xf, x2, ehi_t, elo_t, e2, emb)

    quant_out = jnp.transpose(quant.reshape(B, H, W, C), (0, 3, 1, 2))
    quantize_loss = (loss[0, 0] / (n * C)) * 2.0
    indices = idx.reshape(B, H, W)
    return (quant_out, quantize_loss, indices)


# speedup vs baseline: 1.3097x; 1.3097x over previous
import jax
import jax.numpy as jnp
from jax.experimental import pallas as pl
from jax.experimental.pallas import tpu as pltpu
from jax.experimental.pallas import tpu_sc as plsc

CODEBOOK = 8192
DIM = 32
TILE = 512
CHUNK = 4096
NCHUNK = CODEBOOK // CHUNK
N_TOKENS = 16384
GATHER_WINDOW = 128


def _vq_tile_kernel(x_ref, x2_ref, ehi_ref, elo_ref, e2_ref,
                    idx_ref, loss_ref):
    xt = x_ref[...]
    x_bf = xt.astype(jnp.bfloat16)
    x2 = x2_ref[...]

    v = jnp.full((TILE, 1), jnp.inf, jnp.float32)
    vloss = jnp.zeros((TILE, 1), jnp.float32)
    ix = jnp.zeros((TILE, 1), jnp.int32)
    for c in range(NCHUNK):
        s = slice(c * CHUNK, (c + 1) * CHUNK)
        mm = (jax.lax.dot_general(
                  x_bf, ehi_ref[:, s], (((1,), (0,)), ((), ())),
                  preferred_element_type=jnp.float32)
              + jax.lax.dot_general(
                  x_bf, elo_ref[:, s], (((1,), (0,)), ((), ())),
                  preferred_element_type=jnp.float32))
        d2 = (x2 + e2_ref[:, s]) - 2.0 * mm
        dist = jnp.sqrt(jnp.maximum(d2, 0.0))
        cmin = jnp.min(dist, axis=1, keepdims=True)
        lane = jax.lax.broadcasted_iota(jnp.int32, dist.shape, 1) + c * CHUNK
        cidx = jnp.min(jnp.where(dist == cmin, lane, CODEBOOK),
                       axis=1, keepdims=True)
        take = cmin < v
        v = jnp.where(take, cmin, v)
        vloss = jnp.where(take, cmin, vloss)
        ix = jnp.where(take, cidx, ix)
        v = v.astype(jnp.bfloat16).astype(jnp.float32)

    idx_ref[...] = ix
    part = jnp.sum(vloss * vloss)

    @pl.when(pl.program_id(0) == 0)
    def _():
        loss_ref[0, 0] = 0.0

    loss_ref[0, 0] += part


GATHER_PAD = 128


def _sc_gather(emb_padded, idx_row):
    vector_mesh = plsc.VectorSubcoreMesh(
        core_axis_name="core", subcore_axis_name="subcore")

    @pl.kernel(
        out_type=jax.ShapeDtypeStruct((N_TOKENS, GATHER_PAD),
                                      emb_padded.dtype),
        mesh=vector_mesh)
    def gather_kernel(emb_hbm, i_hbm, o_hbm):
        def body(i_vmem, o_vmem):
            pltpu.sync_copy(emb_hbm.at[i_vmem.at[0]], o_vmem)

        pltpu.emit_pipeline(
            body,
            grid=(N_TOKENS // GATHER_WINDOW,),
            in_specs=[pl.BlockSpec((1, GATHER_WINDOW),
                                   index_map=lambda i: (0, i))],
            out_specs=[pl.BlockSpec((GATHER_WINDOW, GATHER_PAD),
                                    index_map=lambda i: (i, 0))],
            core_axis_name=("core", "subcore"),
            dimension_semantics=(pltpu.PARALLEL,),
        )(i_hbm, o_hbm)

    return gather_kernel(emb_padded, idx_row)


@jax.jit
def kernel(x, emb):
    B, C, H, W = x.shape
    n = B * H * W
    xf = jnp.transpose(x, (0, 2, 3, 1)).reshape(-1, C)
    x2 = jnp.sum(xf ** 2, axis=1, keepdims=True)
    e2 = jnp.sum(emb ** 2, axis=1)[None, :]
    ehi = emb.astype(jnp.bfloat16)
    elo = (emb - ehi.astype(jnp.float32)).astype(jnp.bfloat16)

    grid = n // TILE
    idx, loss = pl.pallas_call(
        _vq_tile_kernel,
        grid=(grid,),
        in_specs=[
            pl.BlockSpec((TILE, C), lambda i: (i, 0)),
            pl.BlockSpec((TILE, 1), lambda i: (i, 0)),
            pl.BlockSpec((C, CODEBOOK), lambda i: (0, 0)),
            pl.BlockSpec((C, CODEBOOK), lambda i: (0, 0)),
            pl.BlockSpec((1, CODEBOOK), lambda i: (0, 0)),
        ],
        out_specs=[
            pl.BlockSpec((TILE, 1), lambda i: (i, 0)),
            pl.BlockSpec((1, 1), lambda i: (0, 0),
                         memory_space=pltpu.SMEM),
        ],
        out_shape=[
            jax.ShapeDtypeStruct((n, 1), jnp.int32),
            jax.ShapeDtypeStruct((1, 1), jnp.float32),
        ],
    )(xf, x2, ehi.T, elo.T, e2)

    emb_padded = jnp.pad(emb, ((0, 0), (0, GATHER_PAD - C)))
    quant = _sc_gather(emb_padded, idx.reshape(1, n))[:, :C]
    quant_out = jnp.transpose(quant.reshape(B, H, W, C), (0, 3, 1, 2))
    quantize_loss = (loss[0, 0] / (n * C)) * 2.0
    indices = idx.reshape(B, H, W)
    return (quant_out, quantize_loss, indices)


# speedup vs baseline: 1.3169x; 1.0055x over previous
import jax
import jax.numpy as jnp
from jax.experimental import pallas as pl
from jax.experimental.pallas import tpu as pltpu
from jax.experimental.pallas import tpu_sc as plsc

CODEBOOK = 8192
DIM = 32
TILE = 1024
CHUNK = 4096
NCHUNK = CODEBOOK // CHUNK
N_TOKENS = 16384
GATHER_WINDOW = 128


def _vq_tile_kernel(x_ref, x2_ref, ehi_ref, elo_ref, e2_ref,
                    idx_ref, loss_ref):
    xt = x_ref[...]
    x_bf = xt.astype(jnp.bfloat16)
    x2 = x2_ref[...]

    v = jnp.full((TILE, 1), jnp.inf, jnp.float32)
    vloss = jnp.zeros((TILE, 1), jnp.float32)
    ix = jnp.zeros((TILE, 1), jnp.int32)
    for c in range(NCHUNK):
        s = slice(c * CHUNK, (c + 1) * CHUNK)
        mm = (jax.lax.dot_general(
                  x_bf, ehi_ref[:, s], (((1,), (0,)), ((), ())),
                  preferred_element_type=jnp.float32)
              + jax.lax.dot_general(
                  x_bf, elo_ref[:, s], (((1,), (0,)), ((), ())),
                  preferred_element_type=jnp.float32))
        dist = jnp.sqrt(jnp.maximum((x2 + e2_ref[:, s]) - 2.0 * mm, 0.0))
        cmin = jnp.min(dist, axis=1, keepdims=True)
        lane = jax.lax.broadcasted_iota(jnp.int32, dist.shape, 1) + c * CHUNK
        cidx = jnp.min(jnp.where(dist == cmin, lane, CODEBOOK),
                       axis=1, keepdims=True)
        take = cmin < v
        v = jnp.where(take, cmin, v)
        vloss = jnp.where(take, cmin, vloss)
        ix = jnp.where(take, cidx, ix)
        v = v.astype(jnp.bfloat16).astype(jnp.float32)

    idx_ref[...] = ix
    part = jnp.sum(vloss * vloss)

    @pl.when(pl.program_id(0) == 0)
    def _():
        loss_ref[0, 0] = 0.0

    loss_ref[0, 0] += part


GATHER_PAD = 128


def _sc_gather(emb_padded, idx_row):
    vector_mesh = plsc.VectorSubcoreMesh(
        core_axis_name="core", subcore_axis_name="subcore")

    @pl.kernel(
        out_type=jax.ShapeDtypeStruct((N_TOKENS, GATHER_PAD),
                                      emb_padded.dtype),
        mesh=vector_mesh)
    def gather_kernel(emb_hbm, i_hbm, o_hbm):
        def body(i_vmem, o_vmem):
            pltpu.sync_copy(emb_hbm.at[i_vmem.at[0]], o_vmem)

        pltpu.emit_pipeline(
            body,
            grid=(N_TOKENS // GATHER_WINDOW,),
            in_specs=[pl.BlockSpec((1, GATHER_WINDOW),
                                   index_map=lambda i: (0, i))],
            out_specs=[pl.BlockSpec((GATHER_WINDOW, GATHER_PAD),
                                    index_map=lambda i: (i, 0))],
            core_axis_name=("core", "subcore"),
            dimension_semantics=(pltpu.PARALLEL,),
        )(i_hbm, o_hbm)

    return gather_kernel(emb_padded, idx_row)


@jax.jit
def kernel(x, emb):
    B, C, H, W = x.shape
    n = B * H * W
    xf = jnp.transpose(x, (0, 2, 3, 1)).reshape(-1, C)
    x2 = jnp.sum(xf ** 2, axis=1, keepdims=True)
    e2 = jnp.sum(emb ** 2, axis=1)[None, :]
    ehi = emb.astype(jnp.bfloat16)
    elo = (emb - ehi.astype(jnp.float32)).astype(jnp.bfloat16)

    grid = n // TILE
    idx, loss = pl.pallas_call(
        _vq_tile_kernel,
        grid=(grid,),
        in_specs=[
            pl.BlockSpec((TILE, C), lambda i: (i, 0)),
            pl.BlockSpec((TILE, 1), lambda i: (i, 0)),
            pl.BlockSpec((C, CODEBOOK), lambda i: (0, 0)),
            pl.BlockSpec((C, CODEBOOK), lambda i: (0, 0)),
            pl.BlockSpec((1, CODEBOOK), lambda i: (0, 0)),
        ],
        out_specs=[
            pl.BlockSpec((TILE, 1), lambda i: (i, 0)),
            pl.BlockSpec((1, 1), lambda i: (0, 0),
                         memory_space=pltpu.SMEM),
        ],
        out_shape=[
            jax.ShapeDtypeStruct((n, 1), jnp.int32),
            jax.ShapeDtypeStruct((1, 1), jnp.float32),
        ],
    )(xf, x2, ehi.T, elo.T, e2)

    emb_padded = jnp.pad(emb, ((0, 0), (0, GATHER_PAD - C)))
    quant = _sc_gather(emb_padded, idx.reshape(1, n))[:, :C]
    quant_out = jnp.transpose(quant.reshape(B, H, W, C), (0, 3, 1, 2))
    quantize_loss = (loss[0, 0] / (n * C)) * 2.0
    indices = idx.reshape(B, H, W)
    return (quant_out, quantize_loss, indices)
